# baseline (speedup 1.0000x reference)
import functools

import jax
import jax.numpy as jnp
from jax import lax
from jax.experimental import pallas as pl
from jax.experimental.pallas import tpu as pltpu
from jax.experimental.pallas import tpu_sc as plsc

NC, NS, L = 2, 16, 16
NW = NC * NS
BATCH = 16384
D = 32
PER_W = BATCH // NW
CHUNK = 128
NCHUNK = PER_W // CHUNK
NGRP = CHUNK // L
EPS = 1e-5

_LN2 = 0.6931471805599453
_SQRT2 = 1.4142135623730951


def _sqrt(x):
    i = lax.bitcast_convert_type(x, jnp.int32)
    y = lax.bitcast_convert_type((i >> 1) + 0x1FBD1DF5, jnp.float32)
    y = 0.5 * (y + x / y)
    y = 0.5 * (y + x / y)
    y = 0.5 * (y + x / y)
    return y


def _log(x):
    i = lax.bitcast_convert_type(x, jnp.int32)
    e = (i >> 23) - 127
    m = lax.bitcast_convert_type((i & 0x007FFFFF) | 0x3F800000, jnp.float32)
    big = m > _SQRT2
    m = jnp.where(big, m * 0.5, m)
    e = (e + big.astype(jnp.int32)).astype(jnp.float32)
    f = m - 1.0
    z = f * f
    p = 7.0376836292e-2
    p = p * f - 1.1514610310e-1
    p = p * f + 1.1676998740e-1
    p = p * f - 1.2420140846e-1
    p = p * f + 1.4249322787e-1
    p = p * f - 1.6668057665e-1
    p = p * f + 2.0000714765e-1
    p = p * f - 2.4999993993e-1
    p = p * f + 3.3333331174e-1
    y = p * f * z - 0.5 * z
    return e * _LN2 + (f + y)


@functools.cache
def _build_poincare_sc():
    mesh = plsc.VectorSubcoreMesh(
        core_axis_name="c", subcore_axis_name="s", num_cores=NC, num_subcores=NS)
    return pl.kernel(
        _poincare_sc_body,
        out_type=jax.ShapeDtypeStruct((BATCH,), jnp.float32),
        mesh=mesh,
        compiler_params=pltpu.CompilerParams(
            use_tc_tiling_on_sc=False, needs_layout_passes=False),
        scratch_types=[
            pltpu.VMEM((NCHUNK, CHUNK), jnp.int32),
            pltpu.VMEM((NCHUNK, CHUNK), jnp.int32),
            pltpu.VMEM((2, CHUNK, D), jnp.float32),
            pltpu.VMEM((2, CHUNK, D), jnp.float32),
            pltpu.VMEM((PER_W,), jnp.float32),
            pltpu.VMEM((2, L), jnp.float32),
            pltpu.SemaphoreType.DMA,
            pltpu.SemaphoreType.DMA,
        ],
    )


def _poincare_sc_body(u_hbm, v_hbm, theta_hbm, ab_hbm, out_hbm,
                      uidx, vidx, eu, ev, outv, abv, sem0, sem1):
    wid = lax.axis_index("s") * NC + lax.axis_index("c")
    base = wid * PER_W
    sems = (sem0, sem1)

    pltpu.sync_copy(ab_hbm, abv)
    for c in range(NCHUNK):
        pltpu.sync_copy(u_hbm.at[pl.ds(base + c * CHUNK, CHUNK)], uidx.at[c])
        pltpu.sync_copy(v_hbm.at[pl.ds(base + c * CHUNK, CHUNK)], vidx.at[c])

    def start(c, buf):
        hu = pltpu.async_copy(theta_hbm.at[uidx.at[c]], eu.at[buf], sems[buf])
        hv = pltpu.async_copy(theta_hbm.at[vidx.at[c]], ev.at[buf], sems[buf])
        return hu, hv

    a = abv[0, :]
    b = abv[1, :]
    iota = jax.lax.iota(jnp.int32, L)

    def compute_chunk(c, buf):
        def group(g, _):
            row_ids = g * L + iota
            uu = jnp.zeros((L,), jnp.float32)
            vv = jnp.zeros((L,), jnp.float32)
            dd = jnp.zeros((L,), jnp.float32)
            for dstep in range(D):
                dim_ids = (iota + dstep) & (D - 1)
                xu = plsc.load_gather(eu.at[buf], [row_ids, dim_ids])
                xv = plsc.load_gather(ev.at[buf], [row_ids, dim_ids])
                uu = uu + xu * xu
                vv = vv + xv * xv
                df = xu - xv
                dd = dd + df * df
            alpha = 1.0 - jnp.minimum(jnp.maximum(uu, 0.0), 1.0 - EPS)
            beta = 1.0 - jnp.minimum(jnp.maximum(vv, 0.0), 1.0 - EPS)
            w = 2.0 * _sqrt(dd + EPS) / (alpha * beta)
            dist = _log(1.0 + w + _sqrt(w * (w + 2.0)))
            outv[pl.ds(c * CHUNK + g * L, L)] = 1.0 / (jnp.exp(dist * a + b) + 1.0)
            return _
        lax.fori_loop(0, NGRP, group, 0, unroll=False)

    handles = start(0, 0)
    for c in range(NCHUNK):
        buf = c % 2
        nxt = start(c + 1, 1 - buf) if c + 1 < NCHUNK else None
        handles[0].wait()
        handles[1].wait()
        compute_chunk(c, buf)
        handles = nxt

    pltpu.sync_copy(outv, out_hbm.at[pl.ds(base, PER_W)])


def kernel(u, v, theta, r, t):
    a = (1.0 / t).astype(jnp.float32)
    b = (-r / t).astype(jnp.float32)
    ab = jnp.stack([jnp.full((L,), a), jnp.full((L,), b)])
    return _build_poincare_sc()(u, v, theta, ab)
